# baseline (speedup 1.0000x reference)
import functools

import jax
import jax.numpy as jnp
from jax import lax
from jax.experimental import pallas as pl
from jax.experimental.pallas import tpu as pltpu
from jax.experimental.pallas import tpu_sc as plsc

B = 32
N = 4096
K = 32
BN = B * N
NW = 32
GPW = BN // NW
CH = 512
NCHUNK = GPW // CH
NB = CH // 16
GW = K * 3
OW = 30

_mesh = plsc.VectorSubcoreMesh(core_axis_name="c", subcore_axis_name="s")


def _rsqrt(v):
    i = lax.bitcast_convert_type(v, jnp.int32)
    i = jnp.int32(0x5F3759DF) - lax.shift_right_logical(i, 1)
    y = lax.bitcast_convert_type(i, jnp.float32)
    for _ in range(3):
        y = y * (1.5 - 0.5 * v * y * y)
    return y


@functools.partial(
    pl.kernel,
    out_type=jax.ShapeDtypeStruct((BN * OW,), jnp.float32),
    mesh=_mesh,
    scratch_types=[
        pltpu.VMEM((CH * GW,), jnp.float32),
        pltpu.VMEM((CH * 3,), jnp.float32),
        pltpu.VMEM((CH * OW,), jnp.float32),
        pltpu.VMEM((512,), jnp.float32),
        pltpu.VMEM((K * 16,), jnp.int32),
    ],
    compiler_params=pltpu.CompilerParams(needs_layout_passes=False),
)
def _pointhop_sc(gx, nc, out, inbuf, cbuf, obuf, acc, tbl):
    wid = lax.axis_index("s") * 2 + lax.axis_index("c")
    lane = jnp.arange(16, dtype=jnp.int32)
    zeros16 = jnp.zeros((16,), jnp.float32)
    ones16 = jnp.ones((16,), jnp.float32)

    for t in range(K):
        kk = lane + t
        kk = jnp.where(kk >= K, kk - K, kk)
        tbl[pl.ds(t * 16, 16)] = lane * GW + kk * 3

    def batch_body(b, carry):
        bsplat = jnp.full((16,), b * (16 * GW), jnp.int32)
        for i in range(32):
            acc[pl.ds(i * 16, 16)] = zeros16
        sxx = zeros16
        syy = zeros16
        szz = zeros16
        for t in range(K):
            ix = tbl[pl.ds(t * 16, 16)] + bsplat
            xs = plsc.load_gather(inbuf, [ix])
            ys = plsc.load_gather(inbuf, [ix + 1])
            zs = plsc.load_gather(inbuf, [ix + 2])
            soff = (jnp.where(xs > 0, 256, 0)
                    + jnp.where(ys > 0, 128, 0)
                    + jnp.where(zs > 0, 64, 0)) + lane
            plsc.addupdate_scatter(acc, [soff], xs)
            plsc.addupdate_scatter(acc, [soff + 16], ys)
            plsc.addupdate_scatter(acc, [soff + 32], zs)
            plsc.addupdate_scatter(acc, [soff + 48], ones16)
            sxx = sxx + xs * xs
            syy = syy + ys * ys
            szz = szz + zs * zs

        obase = lane * OW + b * (16 * OW)

        sums = [zeros16, zeros16, zeros16]
        for c in range(3):
            s = acc[pl.ds(0 * 64 + c * 16, 16)]
            for o in range(1, 8):
                s = s + acc[pl.ds(o * 64 + c * 16, 16)]
            sums[c] = s
        for c, sq in zip(range(3), (sxx, syy, szz)):
            mean = sums[c] * (1.0 / K)
            var = (sq - mean * mean * K) * (1.0 / (K - 1))
            var = jnp.maximum(var, 0.0)
            plsc.store_scatter(obuf, [obase + c], var * _rsqrt(var))

        cb = jnp.full((16,), b * 48, jnp.int32) + lane * 3
        for c in range(3):
            ctr = plsc.load_gather(cbuf, [cb + c])
            plsc.store_scatter(obuf, [obase + 3 + c], ctr)

        for o in range(8):
            cnt = acc[pl.ds(o * 64 + 48, 16)]
            inv = 1.0 / jnp.maximum(cnt, 1.0)
            for c in range(3):
                m = acc[pl.ds(o * 64 + c * 16, 16)] * inv
                plsc.store_scatter(obuf, [obase + 6 + o * 3 + c], m)
        return carry

    def chunk_body(c, carry):
        goff = wid * GPW + c * CH
        pltpu.sync_copy(gx.at[pl.ds(goff * GW, CH * GW)], inbuf)
        pltpu.sync_copy(nc.at[pl.ds(goff * 3, CH * 3)], cbuf)
        lax.fori_loop(0, NB, batch_body, 0, unroll=False)
        pltpu.sync_copy(obuf, out.at[pl.ds(goff * OW, CH * OW)])
        return carry

    lax.fori_loop(0, NCHUNK, chunk_body, 0, unroll=False)


def kernel(group_xyz, new_xyz):
    gx = group_xyz.reshape(-1)
    nc = new_xyz.reshape(-1)
    out = _pointhop_sc(gx, nc)
    return out.reshape(BN, OW)


# speedup vs baseline: 25.6764x; 25.6764x over previous
import functools

import jax
import jax.numpy as jnp
from jax import lax
from jax.experimental import pallas as pl
from jax.experimental.pallas import tpu as pltpu
from jax.experimental.pallas import tpu_sc as plsc

B = 32
N = 4096
K = 32
BN = B * N
NW = 32
CH = 512
NCHUNK = N // CH
NB = CH // 16
OW = 30

_mesh = plsc.VectorSubcoreMesh(core_axis_name="c", subcore_axis_name="s")


def _rsqrt(v):
    i = lax.bitcast_convert_type(v, jnp.int32)
    i = jnp.int32(0x5F3759DF) - lax.shift_right_logical(i, 1)
    y = lax.bitcast_convert_type(i, jnp.float32)
    for _ in range(3):
        y = y * (1.5 - 0.5 * v * y * y)
    return y


@functools.partial(
    pl.kernel,
    out_type=jax.ShapeDtypeStruct((OW, BN), jnp.float32),
    mesh=_mesh,
    scratch_types=[
        pltpu.VMEM((3, K, CH), jnp.float32),
        pltpu.VMEM((3, CH), jnp.float32),
        pltpu.VMEM((OW, CH), jnp.float32),
        pltpu.VMEM((512,), jnp.float32),
    ],
    compiler_params=pltpu.CompilerParams(needs_layout_passes=False),
)
def _pointhop_sc(gx, nc, out, inbuf, cbuf, obuf, acc):
    wid = lax.axis_index("s") * 2 + lax.axis_index("c")
    lane = jnp.arange(16, dtype=jnp.int32)
    zeros16 = jnp.zeros((16,), jnp.float32)
    ones16 = jnp.ones((16,), jnp.float32)

    def batch_body(b, carry):
        g0 = b * 16
        for i in range(32):
            acc[pl.ds(i * 16, 16)] = zeros16
        sxx = zeros16
        syy = zeros16
        szz = zeros16
        for t in range(K):
            xs = inbuf[0, t, pl.ds(g0, 16)]
            ys = inbuf[1, t, pl.ds(g0, 16)]
            zs = inbuf[2, t, pl.ds(g0, 16)]
            soff = (jnp.where(xs > 0, 256, 0)
                    + jnp.where(ys > 0, 128, 0)
                    + jnp.where(zs > 0, 64, 0)) + lane
            plsc.addupdate_scatter(acc, [soff], xs)
            plsc.addupdate_scatter(acc, [soff + 16], ys)
            plsc.addupdate_scatter(acc, [soff + 32], zs)
            plsc.addupdate_scatter(acc, [soff + 48], ones16)
            sxx = sxx + xs * xs
            syy = syy + ys * ys
            szz = szz + zs * zs

        for c, sq in zip(range(3), (sxx, syy, szz)):
            s = acc[pl.ds(0 * 64 + c * 16, 16)]
            for o in range(1, 8):
                s = s + acc[pl.ds(o * 64 + c * 16, 16)]
            mean = s * (1.0 / K)
            var = (sq - mean * mean * K) * (1.0 / (K - 1))
            var = jnp.maximum(var, 0.0)
            obuf[c, pl.ds(g0, 16)] = var * _rsqrt(var)

        for c in range(3):
            obuf[3 + c, pl.ds(g0, 16)] = cbuf[c, pl.ds(g0, 16)]

        for o in range(8):
            cnt = acc[pl.ds(o * 64 + 48, 16)]
            inv = 1.0 / jnp.maximum(cnt, 1.0)
            for c in range(3):
                obuf[6 + o * 3 + c, pl.ds(g0, 16)] = (
                    acc[pl.ds(o * 64 + c * 16, 16)] * inv)
        return carry

    def chunk_body(c, carry):
        n0 = c * CH
        pltpu.sync_copy(gx.at[wid, :, :, pl.ds(n0, CH)], inbuf)
        pltpu.sync_copy(nc.at[:, wid, pl.ds(n0, CH)], cbuf)
        lax.fori_loop(0, NB, batch_body, 0, unroll=False)
        pltpu.sync_copy(obuf, out.at[:, pl.ds(wid * N + n0, CH)])
        return carry

    lax.fori_loop(0, NCHUNK, chunk_body, 0, unroll=False)


def kernel(group_xyz, new_xyz):
    gxt = jnp.transpose(group_xyz, (0, 3, 2, 1))
    nct = jnp.transpose(new_xyz, (2, 0, 1))
    out = _pointhop_sc(gxt, nct)
    return out.T


# speedup vs baseline: 36.6737x; 1.4283x over previous
import functools

import jax
import jax.numpy as jnp
from jax import lax
from jax.experimental import pallas as pl
from jax.experimental.pallas import tpu as pltpu
from jax.experimental.pallas import tpu_sc as plsc

B = 32
N = 4096
K = 32
BN = B * N
NW = 32
CH = 512
NCHUNK = N // CH
NB = CH // 16
OW = 30

_mesh = plsc.VectorSubcoreMesh(core_axis_name="c", subcore_axis_name="s")


def _rsqrt(v):
    i = lax.bitcast_convert_type(v, jnp.int32)
    i = jnp.int32(0x5F3759DF) - lax.shift_right_logical(i, 1)
    y = lax.bitcast_convert_type(i, jnp.float32)
    for _ in range(3):
        y = y * (1.5 - 0.5 * v * y * y)
    return y


@functools.partial(
    pl.kernel,
    out_type=jax.ShapeDtypeStruct((OW, BN), jnp.float32),
    mesh=_mesh,
    scratch_types=[
        pltpu.VMEM((3, K, CH), jnp.float32),
        pltpu.VMEM((3, CH), jnp.float32),
        pltpu.VMEM((OW, CH), jnp.float32),
        pltpu.VMEM((512,), jnp.float32),
    ],
    compiler_params=pltpu.CompilerParams(needs_layout_passes=False),
)
def _pointhop_sc(gx, nc, out, inbuf, cbuf, obuf, acc):
    wid = lax.axis_index("s") * 2 + lax.axis_index("c")
    lane = jnp.arange(16, dtype=jnp.int32)
    zeros16 = jnp.zeros((16,), jnp.float32)
    ones16 = jnp.ones((16,), jnp.float32)

    lane256 = lane + 256
    acc_y = acc.at[pl.ds(16, 496)]
    acc_z = acc.at[pl.ds(32, 480)]
    acc_n = acc.at[pl.ds(48, 464)]

    def batch_body(b, carry):
        g0 = b * 16
        for i in range(32):
            acc[pl.ds(i * 16, 16)] = zeros16

        @plsc.parallel_loop(0, K, 1, unroll=8,
                            carry=(zeros16, zeros16, zeros16))
        def sums_sq(t, csum):
            sxx, syy, szz = csum
            xs = inbuf[0, t, pl.ds(g0, 16)]
            ys = inbuf[1, t, pl.ds(g0, 16)]
            zs = inbuf[2, t, pl.ds(g0, 16)]
            soff = (jnp.where(xs > 0, lane256, lane)
                    + jnp.where(ys > 0, 128, 0)
                    + jnp.where(zs > 0, 64, 0))
            plsc.addupdate_scatter(acc, [soff], xs)
            plsc.addupdate_scatter(acc_y, [soff], ys)
            plsc.addupdate_scatter(acc_z, [soff], zs)
            plsc.addupdate_scatter(acc_n, [soff], ones16)
            return (sxx + xs * xs, syy + ys * ys, szz + zs * zs)

        sxx, syy, szz = sums_sq

        for c, sq in zip(range(3), (sxx, syy, szz)):
            s = acc[pl.ds(0 * 64 + c * 16, 16)]
            for o in range(1, 8):
                s = s + acc[pl.ds(o * 64 + c * 16, 16)]
            mean = s * (1.0 / K)
            var = (sq - mean * mean * K) * (1.0 / (K - 1))
            var = jnp.maximum(var, 0.0)
            obuf[c, pl.ds(g0, 16)] = var * _rsqrt(var)

        for c in range(3):
            obuf[3 + c, pl.ds(g0, 16)] = cbuf[c, pl.ds(g0, 16)]

        for o in range(8):
            cnt = acc[pl.ds(o * 64 + 48, 16)]
            inv = 1.0 / jnp.maximum(cnt, 1.0)
            for c in range(3):
                obuf[6 + o * 3 + c, pl.ds(g0, 16)] = (
                    acc[pl.ds(o * 64 + c * 16, 16)] * inv)
        return carry

    def chunk_body(c, carry):
        n0 = c * CH
        pltpu.sync_copy(gx.at[wid, :, :, pl.ds(n0, CH)], inbuf)
        pltpu.sync_copy(nc.at[:, wid, pl.ds(n0, CH)], cbuf)
        lax.fori_loop(0, NB, batch_body, 0, unroll=False)
        pltpu.sync_copy(obuf, out.at[:, pl.ds(wid * N + n0, CH)])
        return carry

    lax.fori_loop(0, NCHUNK, chunk_body, 0, unroll=False)


def kernel(group_xyz, new_xyz):
    gxt = jnp.transpose(group_xyz, (0, 3, 2, 1))
    nct = jnp.transpose(new_xyz, (2, 0, 1))
    out = _pointhop_sc(gxt, nct)
    return out.T


# speedup vs baseline: 43.3238x; 1.1813x over previous
import functools

import jax
import jax.numpy as jnp
from jax import lax
from jax.experimental import pallas as pl
from jax.experimental.pallas import tpu as pltpu
from jax.experimental.pallas import tpu_sc as plsc

B = 32
N = 4096
K = 32
BN = B * N
NW = 32
CH = 512
NCHUNK = N // CH
NP = NCHUNK // 2
NB = CH // 16
OW = 30

_mesh = plsc.VectorSubcoreMesh(core_axis_name="c", subcore_axis_name="s")


def _rsqrt(v):
    i = lax.bitcast_convert_type(v, jnp.int32)
    i = jnp.int32(0x5F3759DF) - lax.shift_right_logical(i, 1)
    y = lax.bitcast_convert_type(i, jnp.float32)
    for _ in range(3):
        y = y * (1.5 - 0.5 * v * y * y)
    return y


@functools.partial(
    pl.kernel,
    out_type=jax.ShapeDtypeStruct((OW, BN), jnp.float32),
    mesh=_mesh,
    scratch_types=[
        pltpu.VMEM((3, K, CH), jnp.float32),
        pltpu.VMEM((3, K, CH), jnp.float32),
        pltpu.VMEM((OW, CH), jnp.float32),
        pltpu.VMEM((512,), jnp.float32),
        pltpu.SemaphoreType.DMA,
        pltpu.SemaphoreType.DMA,
        pltpu.SemaphoreType.DMA,
    ],
    compiler_params=pltpu.CompilerParams(needs_layout_passes=False),
)
def _pointhop_sc(gx, nc, out, in0, in1, obuf, acc, sin0, sin1, scen):
    wid = lax.axis_index("s") * 2 + lax.axis_index("c")
    lane = jnp.arange(16, dtype=jnp.int32)
    zeros16 = jnp.zeros((16,), jnp.float32)
    ones16 = jnp.ones((16,), jnp.float32)
    lane256 = lane + 256
    acc_y = acc.at[pl.ds(16, 496)]
    acc_z = acc.at[pl.ds(32, 480)]
    acc_n = acc.at[pl.ds(48, 464)]

    def in_copy(c, ibuf, sem):
        return pltpu.make_async_copy(
            gx.at[wid, :, :, pl.ds(c * CH, CH)], ibuf, sem)

    def batch_body_for(ibuf):
        def batch_body(b, carry):
            g0 = b * 16
            for i in range(32):
                acc[pl.ds(i * 16, 16)] = zeros16

            @plsc.parallel_loop(0, K, 1, unroll=8,
                                carry=(zeros16, zeros16, zeros16))
            def sums_sq(t, csum):
                sxx, syy, szz = csum
                xs = ibuf[0, t, pl.ds(g0, 16)]
                ys = ibuf[1, t, pl.ds(g0, 16)]
                zs = ibuf[2, t, pl.ds(g0, 16)]
                soff = (jnp.where(xs > 0, lane256, lane)
                        + jnp.where(ys > 0, 128, 0)
                        + jnp.where(zs > 0, 64, 0))
                plsc.addupdate_scatter(acc, [soff], xs)
                plsc.addupdate_scatter(acc_y, [soff], ys)
                plsc.addupdate_scatter(acc_z, [soff], zs)
                plsc.addupdate_scatter(acc_n, [soff], ones16)
                return (sxx + xs * xs, syy + ys * ys, szz + zs * zs)

            sxx, syy, szz = sums_sq

            for c, sq in zip(range(3), (sxx, syy, szz)):
                s = acc[pl.ds(0 * 64 + c * 16, 16)]
                for o in range(1, 8):
                    s = s + acc[pl.ds(o * 64 + c * 16, 16)]
                mean = s * (1.0 / K)
                var = (sq - mean * mean * K) * (1.0 / (K - 1))
                var = jnp.maximum(var, 0.0)
                obuf[c, pl.ds(g0, 16)] = var * _rsqrt(var)

            for o in range(8):
                cnt = acc[pl.ds(o * 64 + 48, 16)]
                inv = 1.0 / jnp.maximum(cnt, 1.0)
                for c in range(3):
                    obuf[6 + o * 3 + c, pl.ds(g0, 16)] = (
                        acc[pl.ds(o * 64 + c * 16, 16)] * inv)
            return carry
        return batch_body

    body0 = batch_body_for(in0)
    body1 = batch_body_for(in1)

    def do_chunk(c, ibuf, sem, body, prefetch):
        cen = pltpu.async_copy(nc.at[:, wid, pl.ds(c * CH, CH)],
                               obuf.at[pl.ds(3, 3), :], scen)
        if prefetch is not None:
            prefetch()
        in_copy(c, ibuf, sem).wait()
        lax.fori_loop(0, NB, body, 0, unroll=False)
        cen.wait()
        pltpu.sync_copy(obuf, out.at[:, pl.ds(wid * N + c * CH, CH)])

    def pair_body(p, carry):
        c0 = p * 2
        do_chunk(c0, in0, sin0, body0,
                 lambda: in_copy(c0 + 1, in1, sin1).start())
        @pl.when(p + 1 < NP)
        def _():
            in_copy(c0 + 2, in0, sin0).start()
        do_chunk(c0 + 1, in1, sin1, body1, None)
        return carry

    in_copy(0, in0, sin0).start()
    lax.fori_loop(0, NP, pair_body, 0, unroll=False)


def kernel(group_xyz, new_xyz):
    gxt = jnp.transpose(group_xyz, (0, 3, 2, 1))
    nct = jnp.transpose(new_xyz, (2, 0, 1))
    out = _pointhop_sc(gxt, nct)
    return out.T


# speedup vs baseline: 68.9125x; 1.5906x over previous
import functools

import jax
import jax.numpy as jnp
from jax import lax
from jax.experimental import pallas as pl
from jax.experimental.pallas import tpu as pltpu
from jax.experimental.pallas import tpu_sc as plsc

B = 32
N = 4096
K = 32
BN = B * N
NW = 32
CH = 512
NCHUNK = N // CH
NP = NCHUNK // 2
NB = CH // 16
OW = 30

_mesh = plsc.VectorSubcoreMesh(core_axis_name="c", subcore_axis_name="s")


def _rsqrt(v):
    i = lax.bitcast_convert_type(v, jnp.int32)
    i = jnp.int32(0x5F3759DF) - lax.shift_right_logical(i, 1)
    y = lax.bitcast_convert_type(i, jnp.float32)
    for _ in range(3):
        y = y * (1.5 - 0.5 * v * y * y)
    return y


@functools.partial(
    pl.kernel,
    out_type=jax.ShapeDtypeStruct((OW, BN), jnp.float32),
    mesh=_mesh,
    scratch_types=[
        pltpu.VMEM((3, K, CH), jnp.float32),
        pltpu.VMEM((3, K, CH), jnp.float32),
        pltpu.VMEM((OW, CH), jnp.float32),
        pltpu.VMEM((512,), jnp.float32),
        pltpu.SemaphoreType.DMA,
        pltpu.SemaphoreType.DMA,
        pltpu.SemaphoreType.DMA,
    ],
    compiler_params=pltpu.CompilerParams(needs_layout_passes=False),
)
def _pointhop_sc(gx, nc, out, in0, in1, obuf, acc, sin0, sin1, scen):
    wid = lax.axis_index("s") * 2 + lax.axis_index("c")
    lane = jnp.arange(16, dtype=jnp.int32)
    zeros16 = jnp.zeros((16,), jnp.float32)
    ones16 = jnp.ones((16,), jnp.float32)
    lane256 = lane + 256
    acc_y = acc.at[pl.ds(16, 496)]
    acc_z = acc.at[pl.ds(32, 480)]
    acc_n = acc.at[pl.ds(48, 464)]

    def in_copy(c, ibuf, sem):
        return pltpu.make_async_copy(
            gx.at[wid, :, :, pl.ds(c * CH, CH)], ibuf, sem)

    def batch_body_for(ibuf):
        def batch_body(b, carry):
            g0 = b * 16
            for i in range(32):
                acc[pl.ds(i * 16, 16)] = zeros16

            z6 = (zeros16,) * 6

            @plsc.parallel_loop(0, K, 1, unroll=8, carry=z6)
            def sums_sq(t, csum):
                sx, sy, sz, sxx, syy, szz = csum
                xs = ibuf[0, t, pl.ds(g0, 16)]
                ys = ibuf[1, t, pl.ds(g0, 16)]
                zs = ibuf[2, t, pl.ds(g0, 16)]
                soff = (jnp.where(xs > 0, lane256, lane)
                        + jnp.where(ys > 0, 128, 0)
                        + jnp.where(zs > 0, 64, 0))
                plsc.addupdate_scatter(acc, [soff], xs)
                plsc.addupdate_scatter(acc_y, [soff], ys)
                plsc.addupdate_scatter(acc_z, [soff], zs)
                plsc.addupdate_scatter(acc_n, [soff], ones16)
                return (sx + xs, sy + ys, sz + zs,
                        sxx + xs * xs, syy + ys * ys, szz + zs * zs)

            sx, sy, sz, sxx, syy, szz = sums_sq

            for c, s, sq in zip(range(3), (sx, sy, sz), (sxx, syy, szz)):
                mean = s * (1.0 / K)
                var = (sq - mean * mean * K) * (1.0 / (K - 1))
                var = jnp.maximum(var, 0.0)
                obuf[c, pl.ds(g0, 16)] = var * _rsqrt(var)

            @plsc.parallel_loop(0, 8, 1, unroll=4)
            def _(o):
                o64 = o * 64
                cnt = acc[pl.ds(o64 + 48, 16)]
                inv = 1.0 / jnp.maximum(cnt, 1.0)
                ox = acc[pl.ds(o64, 16)] * inv
                oy = acc[pl.ds(o64 + 16, 16)] * inv
                oz = acc[pl.ds(o64 + 32, 16)] * inv
                o3 = 6 + o * 3
                obuf[o3, pl.ds(g0, 16)] = ox
                obuf[o3 + 1, pl.ds(g0, 16)] = oy
                obuf[o3 + 2, pl.ds(g0, 16)] = oz
            return carry
        return batch_body

    body0 = batch_body_for(in0)
    body1 = batch_body_for(in1)

    def do_chunk(c, ibuf, sem, body, prefetch):
        cen = pltpu.async_copy(nc.at[:, wid, pl.ds(c * CH, CH)],
                               obuf.at[pl.ds(3, 3), :], scen)
        if prefetch is not None:
            prefetch()
        in_copy(c, ibuf, sem).wait()
        lax.fori_loop(0, NB, body, 0, unroll=False)
        cen.wait()
        pltpu.sync_copy(obuf, out.at[:, pl.ds(wid * N + c * CH, CH)])

    def pair_body(p, carry):
        c0 = p * 2
        do_chunk(c0, in0, sin0, body0,
                 lambda: in_copy(c0 + 1, in1, sin1).start())
        @pl.when(p + 1 < NP)
        def _():
            in_copy(c0 + 2, in0, sin0).start()
        do_chunk(c0 + 1, in1, sin1, body1, None)
        return carry

    in_copy(0, in0, sin0).start()
    lax.fori_loop(0, NP, pair_body, 0, unroll=False)


def kernel(group_xyz, new_xyz):
    gxt = jnp.transpose(group_xyz, (0, 3, 2, 1))
    nct = jnp.transpose(new_xyz, (2, 0, 1))
    out = _pointhop_sc(gxt, nct)
    return out.T


# speedup vs baseline: 76.8482x; 1.1152x over previous
import functools

import jax
import jax.numpy as jnp
from jax import lax
from jax.experimental import pallas as pl
from jax.experimental.pallas import tpu as pltpu
from jax.experimental.pallas import tpu_sc as plsc

B = 32
N = 4096
K = 32
BN = B * N
NW = 32
CH = 256
NCHUNK = N // CH
NP = NCHUNK // 2
NB = CH // 16
OW = 30

_mesh = plsc.VectorSubcoreMesh(core_axis_name="c", subcore_axis_name="s")


def _rsqrt(v):
    i = lax.bitcast_convert_type(v, jnp.int32)
    i = jnp.int32(0x5F3759DF) - lax.shift_right_logical(i, 1)
    y = lax.bitcast_convert_type(i, jnp.float32)
    for _ in range(3):
        y = y * (1.5 - 0.5 * v * y * y)
    return y


@functools.partial(
    pl.kernel,
    out_type=jax.ShapeDtypeStruct((OW, BN), jnp.float32),
    mesh=_mesh,
    scratch_types=[
        pltpu.VMEM((3, K, CH), jnp.float32),
        pltpu.VMEM((3, K, CH), jnp.float32),
        pltpu.VMEM((OW, CH), jnp.float32),
        pltpu.VMEM((OW, CH), jnp.float32),
        pltpu.VMEM((512,), jnp.float32),
        pltpu.SemaphoreType.DMA,
        pltpu.SemaphoreType.DMA,
        pltpu.SemaphoreType.DMA,
        pltpu.SemaphoreType.DMA,
        pltpu.SemaphoreType.DMA,
    ],
    compiler_params=pltpu.CompilerParams(needs_layout_passes=False),
)
def _pointhop_sc(gx, nc, out, in0, in1, ob0, ob1, acc,
                 sin0, sin1, scen, sob0, sob1):
    wid = lax.axis_index("s") * 2 + lax.axis_index("c")
    lane = jnp.arange(16, dtype=jnp.int32)
    zeros16 = jnp.zeros((16,), jnp.float32)
    ones16 = jnp.ones((16,), jnp.float32)
    lane256 = lane + 256
    acc_y = acc.at[pl.ds(16, 496)]
    acc_z = acc.at[pl.ds(32, 480)]
    acc_n = acc.at[pl.ds(48, 464)]

    def in_copy(c, ibuf, sem):
        return pltpu.make_async_copy(
            gx.at[wid, :, :, pl.ds(c * CH, CH)], ibuf, sem)

    def batch_body_for(ibuf, obuf):
        def batch_body(b, carry):
            g0 = b * 16
            z6 = (zeros16,) * 6

            @plsc.parallel_loop(0, K, 1, unroll=8, carry=z6)
            def sums_sq(t, csum):
                sx, sy, sz, sxx, syy, szz = csum
                xs = ibuf[0, t, pl.ds(g0, 16)]
                ys = ibuf[1, t, pl.ds(g0, 16)]
                zs = ibuf[2, t, pl.ds(g0, 16)]
                soff = (jnp.where(xs > 0, lane256, lane)
                        + jnp.where(ys > 0, 128, 0)
                        + jnp.where(zs > 0, 64, 0))
                plsc.addupdate_scatter(acc, [soff], xs)
                plsc.addupdate_scatter(acc_y, [soff], ys)
                plsc.addupdate_scatter(acc_z, [soff], zs)
                plsc.addupdate_scatter(acc_n, [soff], ones16)
                return (sx + xs, sy + ys, sz + zs,
                        sxx + xs * xs, syy + ys * ys, szz + zs * zs)

            sx, sy, sz, sxx, syy, szz = sums_sq

            for c, s, sq in zip(range(3), (sx, sy, sz), (sxx, syy, szz)):
                mean = s * (1.0 / K)
                var = (sq - mean * mean * K) * (1.0 / (K - 1))
                var = jnp.maximum(var, 0.0)
                obuf[c, pl.ds(g0, 16)] = var * _rsqrt(var)

            @plsc.parallel_loop(0, 8, 1, unroll=4)
            def _(o):
                o64 = o * 64
                cnt = acc[pl.ds(o64 + 48, 16)]
                inv = 1.0 / jnp.maximum(cnt, 1.0)
                ox = acc[pl.ds(o64, 16)] * inv
                oy = acc[pl.ds(o64 + 16, 16)] * inv
                oz = acc[pl.ds(o64 + 32, 16)] * inv
                acc[pl.ds(o64, 16)] = zeros16
                acc[pl.ds(o64 + 16, 16)] = zeros16
                acc[pl.ds(o64 + 32, 16)] = zeros16
                acc[pl.ds(o64 + 48, 16)] = zeros16
                o3 = 6 + o * 3
                obuf[o3, pl.ds(g0, 16)] = ox
                obuf[o3 + 1, pl.ds(g0, 16)] = oy
                obuf[o3 + 2, pl.ds(g0, 16)] = oz
            return carry
        return batch_body

    body0 = batch_body_for(in0, ob0)
    body1 = batch_body_for(in1, ob1)

    def out_copy(c, obuf, sem):
        return pltpu.make_async_copy(
            obuf, out.at[:, pl.ds(wid * N + c * CH, CH)], sem)

    def do_chunk(c, ibuf, sem, obuf, osem, body, prefetch, first):
        @pl.when(jnp.logical_not(first))
        def _():
            out_copy(c, obuf, osem).wait()
        cen = pltpu.async_copy(nc.at[:, wid, pl.ds(c * CH, CH)],
                               obuf.at[pl.ds(3, 3), :], scen)
        if prefetch is not None:
            prefetch()
        in_copy(c, ibuf, sem).wait()
        lax.fori_loop(0, NB, body, 0, unroll=False)
        cen.wait()
        out_copy(c, obuf, osem).start()

    def pair_body(p, carry):
        c0 = p * 2
        first = p == 0
        do_chunk(c0, in0, sin0, ob0, sob0, body0,
                 lambda: in_copy(c0 + 1, in1, sin1).start(), first)
        @pl.when(p + 1 < NP)
        def _():
            in_copy(c0 + 2, in0, sin0).start()
        do_chunk(c0 + 1, in1, sin1, ob1, sob1, body1, None, first)
        return carry

    for i in range(32):
        acc[pl.ds(i * 16, 16)] = zeros16
    in_copy(0, in0, sin0).start()
    lax.fori_loop(0, NP, pair_body, 0, unroll=False)
    out_copy(NCHUNK - 2, ob0, sob0).wait()
    out_copy(NCHUNK - 1, ob1, sob1).wait()


def kernel(group_xyz, new_xyz):
    gxt = jnp.transpose(group_xyz, (0, 3, 2, 1))
    nct = jnp.transpose(new_xyz, (2, 0, 1))
    out = _pointhop_sc(gxt, nct)
    return out.T


# speedup vs baseline: 77.9138x; 1.0139x over previous
import functools

import jax
import jax.numpy as jnp
from jax import lax
from jax.experimental import pallas as pl
from jax.experimental.pallas import tpu as pltpu
from jax.experimental.pallas import tpu_sc as plsc

B = 32
N = 4096
K = 32
BN = B * N
NW = 32
CH = 256
NCHUNK = N // CH
NP = NCHUNK // 2
NB = CH // 16
OW = 30

_mesh = plsc.VectorSubcoreMesh(core_axis_name="c", subcore_axis_name="s")


def _rsqrt(v):
    vh = v * 0.5
    i = lax.bitcast_convert_type(v, jnp.int32)
    i = jnp.int32(0x5F3759DF) - lax.shift_right_logical(i, 1)
    y = lax.bitcast_convert_type(i, jnp.float32)
    for _ in range(2):
        y = y * (1.5 - vh * y * y)
    return y


@functools.partial(
    pl.kernel,
    out_type=jax.ShapeDtypeStruct((OW, BN), jnp.float32),
    mesh=_mesh,
    scratch_types=[
        pltpu.VMEM((3, K, CH), jnp.float32),
        pltpu.VMEM((3, K, CH), jnp.float32),
        pltpu.VMEM((OW, CH), jnp.float32),
        pltpu.VMEM((OW, CH), jnp.float32),
        pltpu.VMEM((512,), jnp.float32),
        pltpu.SemaphoreType.DMA,
        pltpu.SemaphoreType.DMA,
        pltpu.SemaphoreType.DMA,
        pltpu.SemaphoreType.DMA,
        pltpu.SemaphoreType.DMA,
    ],
    compiler_params=pltpu.CompilerParams(needs_layout_passes=False),
)
def _pointhop_sc(gx, nc, out, in0, in1, ob0, ob1, acc,
                 sin0, sin1, scen, sob0, sob1):
    wid = lax.axis_index("s") * 2 + lax.axis_index("c")
    lane = jnp.arange(16, dtype=jnp.int32)
    zeros16 = jnp.zeros((16,), jnp.float32)
    ones16 = jnp.ones((16,), jnp.float32)
    lane256 = lane + 256
    acc_y = acc.at[pl.ds(16, 496)]
    acc_z = acc.at[pl.ds(32, 480)]
    acc_n = acc.at[pl.ds(48, 464)]

    def in_copy(c, ibuf, sem):
        return pltpu.make_async_copy(
            gx.at[wid, :, :, pl.ds(c * CH, CH)], ibuf, sem)

    def batch_body_for(ibuf, obuf):
        def batch_body(b, carry):
            g0 = b * 16
            z3 = (zeros16,) * 3

            @plsc.parallel_loop(0, K, 1, unroll=8, carry=z3)
            def sums_sq(t, csum):
                sxx, syy, szz = csum
                xs = ibuf[0, t, pl.ds(g0, 16)]
                ys = ibuf[1, t, pl.ds(g0, 16)]
                zs = ibuf[2, t, pl.ds(g0, 16)]
                soff = (jnp.where(xs > 0, lane256, lane)
                        + jnp.where(ys > 0, 128, 0)
                        + jnp.where(zs > 0, 64, 0))
                plsc.addupdate_scatter(acc, [soff], xs)
                plsc.addupdate_scatter(acc_y, [soff], ys)
                plsc.addupdate_scatter(acc_z, [soff], zs)
                plsc.addupdate_scatter(acc_n, [soff], ones16)
                return (sxx + xs * xs, syy + ys * ys, szz + zs * zs)

            sxx, syy, szz = sums_sq

            @plsc.parallel_loop(0, 8, 1, unroll=4, carry=z3)
            def totals(o, tot):
                tx, ty, tz = tot
                o64 = o * 64
                cnt = acc[pl.ds(o64 + 48, 16)]
                inv = 1.0 / jnp.maximum(cnt, 1.0)
                bx = acc[pl.ds(o64, 16)]
                by = acc[pl.ds(o64 + 16, 16)]
                bz = acc[pl.ds(o64 + 32, 16)]
                acc[pl.ds(o64, 16)] = zeros16
                acc[pl.ds(o64 + 16, 16)] = zeros16
                acc[pl.ds(o64 + 32, 16)] = zeros16
                acc[pl.ds(o64 + 48, 16)] = zeros16
                o3 = 6 + o * 3
                obuf[o3, pl.ds(g0, 16)] = bx * inv
                obuf[o3 + 1, pl.ds(g0, 16)] = by * inv
                obuf[o3 + 2, pl.ds(g0, 16)] = bz * inv
                return (tx + bx, ty + by, tz + bz)

            for c, s, sq in zip(range(3), totals, (sxx, syy, szz)):
                var = sq * (1.0 / (K - 1)) - (s * s) * (1.0 / (K * (K - 1)))
                var = jnp.maximum(var, 0.0)
                obuf[c, pl.ds(g0, 16)] = var * _rsqrt(var)
            return carry
        return batch_body

    body0 = batch_body_for(in0, ob0)
    body1 = batch_body_for(in1, ob1)

    def out_copy(c, obuf, sem):
        return pltpu.make_async_copy(
            obuf, out.at[:, pl.ds(wid * N + c * CH, CH)], sem)

    def do_chunk(c, ibuf, sem, obuf, osem, body, prefetch, first):
        @pl.when(jnp.logical_not(first))
        def _():
            out_copy(c, obuf, osem).wait()
        cen = pltpu.async_copy(nc.at[:, wid, pl.ds(c * CH, CH)],
                               obuf.at[pl.ds(3, 3), :], scen)
        if prefetch is not None:
            prefetch()
        in_copy(c, ibuf, sem).wait()
        lax.fori_loop(0, NB, body, 0, unroll=False)
        cen.wait()
        out_copy(c, obuf, osem).start()

    def pair_body(p, carry):
        c0 = p * 2
        first = p == 0
        do_chunk(c0, in0, sin0, ob0, sob0, body0,
                 lambda: in_copy(c0 + 1, in1, sin1).start(), first)
        @pl.when(p + 1 < NP)
        def _():
            in_copy(c0 + 2, in0, sin0).start()
        do_chunk(c0 + 1, in1, sin1, ob1, sob1, body1, None, first)
        return carry

    for i in range(32):
        acc[pl.ds(i * 16, 16)] = zeros16
    in_copy(0, in0, sin0).start()
    lax.fori_loop(0, NP, pair_body, 0, unroll=False)
    out_copy(NCHUNK - 2, ob0, sob0).wait()
    out_copy(NCHUNK - 1, ob1, sob1).wait()


def kernel(group_xyz, new_xyz):
    gxt = jnp.transpose(group_xyz, (0, 3, 2, 1))
    nct = jnp.transpose(new_xyz, (2, 0, 1))
    out = _pointhop_sc(gxt, nct)
    return out.T


# speedup vs baseline: 86.3369x; 1.1081x over previous
import functools

import jax
import jax.numpy as jnp
from jax import lax
from jax.experimental import pallas as pl
from jax.experimental.pallas import tpu as pltpu
from jax.experimental.pallas import tpu_sc as plsc

B = 32
N = 4096
K = 32
BN = B * N
NW = 32
CH = 256
NCHUNK = N // CH
NP = NCHUNK // 2
NB = CH // 16
OW = 30

_mesh = plsc.VectorSubcoreMesh(core_axis_name="c", subcore_axis_name="s")


def _rsqrt(v):
    vh = v * 0.5
    i = lax.bitcast_convert_type(v, jnp.int32)
    i = jnp.int32(0x5F3759DF) - lax.shift_right_logical(i, 1)
    y = lax.bitcast_convert_type(i, jnp.float32)
    for _ in range(2):
        y = y * (1.5 - vh * y * y)
    return y


@functools.partial(
    pl.kernel,
    out_type=jax.ShapeDtypeStruct((OW, BN), jnp.float32),
    mesh=_mesh,
    scratch_types=[
        pltpu.VMEM((3, K, CH), jnp.float32),
        pltpu.VMEM((3, K, CH), jnp.float32),
        pltpu.VMEM((OW, CH), jnp.float32),
        pltpu.VMEM((OW, CH), jnp.float32),
        pltpu.VMEM((512,), jnp.float32),
        pltpu.SemaphoreType.DMA,
        pltpu.SemaphoreType.DMA,
        pltpu.SemaphoreType.DMA,
        pltpu.SemaphoreType.DMA,
        pltpu.SemaphoreType.DMA,
    ],
    compiler_params=pltpu.CompilerParams(needs_layout_passes=False),
)
def _pointhop_sc(gx, nc, out, in0, in1, ob0, ob1, acc,
                 sin0, sin1, scen, sob0, sob1):
    wid = lax.axis_index("s") * 2 + lax.axis_index("c")
    lane = jnp.arange(16, dtype=jnp.int32)
    zeros16 = jnp.zeros((16,), jnp.float32)
    ones16 = jnp.ones((16,), jnp.float32)
    lane256 = lane + 256
    acc_y = acc.at[pl.ds(16, 496)]
    acc_z = acc.at[pl.ds(32, 480)]
    acc_n = acc.at[pl.ds(48, 464)]

    def in_copy(c, ibuf, sem):
        return pltpu.make_async_copy(
            gx.at[wid, :, :, pl.ds(c * CH, CH)], ibuf, sem)

    def batch_body_for(ibuf, obuf):
        def batch_body(b, carry):
            g0 = b * 16
            z3 = (zeros16,) * 3

            @plsc.parallel_loop(0, K, 1, unroll=16, carry=z3)
            def sums_sq(t, csum):
                sxx, syy, szz = csum
                xs = ibuf[0, t, pl.ds(g0, 16)]
                ys = ibuf[1, t, pl.ds(g0, 16)]
                zs = ibuf[2, t, pl.ds(g0, 16)]
                soff = (jnp.where(xs > 0, lane256, lane)
                        + jnp.where(ys > 0, 128, 0)
                        + jnp.where(zs > 0, 64, 0))
                plsc.addupdate_scatter(acc, [soff], xs)
                plsc.addupdate_scatter(acc_y, [soff], ys)
                plsc.addupdate_scatter(acc_z, [soff], zs)
                plsc.addupdate_scatter(acc_n, [soff], ones16)
                return (sxx + xs * xs, syy + ys * ys, szz + zs * zs)

            sxx, syy, szz = sums_sq

            @plsc.parallel_loop(0, 8, 1, unroll=8, carry=z3)
            def totals(o, tot):
                tx, ty, tz = tot
                o64 = o * 64
                cnt = acc[pl.ds(o64 + 48, 16)]
                inv = 1.0 / jnp.maximum(cnt, 1.0)
                bx = acc[pl.ds(o64, 16)]
                by = acc[pl.ds(o64 + 16, 16)]
                bz = acc[pl.ds(o64 + 32, 16)]
                acc[pl.ds(o64, 16)] = zeros16
                acc[pl.ds(o64 + 16, 16)] = zeros16
                acc[pl.ds(o64 + 32, 16)] = zeros16
                acc[pl.ds(o64 + 48, 16)] = zeros16
                o3 = 6 + o * 3
                obuf[o3, pl.ds(g0, 16)] = bx * inv
                obuf[o3 + 1, pl.ds(g0, 16)] = by * inv
                obuf[o3 + 2, pl.ds(g0, 16)] = bz * inv
                return (tx + bx, ty + by, tz + bz)

            for c, s, sq in zip(range(3), totals, (sxx, syy, szz)):
                var = sq * (1.0 / (K - 1)) - (s * s) * (1.0 / (K * (K - 1)))
                var = jnp.maximum(var, 0.0)
                obuf[c, pl.ds(g0, 16)] = var * _rsqrt(var)
            return carry
        return batch_body

    body0 = batch_body_for(in0, ob0)
    body1 = batch_body_for(in1, ob1)

    def out_copy(c, obuf, sem):
        return pltpu.make_async_copy(
            obuf, out.at[:, pl.ds(wid * N + c * CH, CH)], sem)

    def do_chunk(c, ibuf, sem, obuf, osem, body, prefetch, first):
        @pl.when(jnp.logical_not(first))
        def _():
            out_copy(c, obuf, osem).wait()
        cen = pltpu.async_copy(nc.at[:, wid, pl.ds(c * CH, CH)],
                               obuf.at[pl.ds(3, 3), :], scen)
        if prefetch is not None:
            prefetch()
        in_copy(c, ibuf, sem).wait()
        lax.fori_loop(0, NB, body, 0, unroll=False)
        cen.wait()
        out_copy(c, obuf, osem).start()

    def pair_body(p, carry):
        c0 = p * 2
        first = p == 0
        do_chunk(c0, in0, sin0, ob0, sob0, body0,
                 lambda: in_copy(c0 + 1, in1, sin1).start(), first)
        @pl.when(p + 1 < NP)
        def _():
            in_copy(c0 + 2, in0, sin0).start()
        do_chunk(c0 + 1, in1, sin1, ob1, sob1, body1, None, first)
        return carry

    for i in range(32):
        acc[pl.ds(i * 16, 16)] = zeros16
    in_copy(0, in0, sin0).start()
    lax.fori_loop(0, NP, pair_body, 0, unroll=False)
    out_copy(NCHUNK - 2, ob0, sob0).wait()
    out_copy(NCHUNK - 1, ob1, sob1).wait()


def kernel(group_xyz, new_xyz):
    gxt = jnp.transpose(group_xyz, (0, 3, 2, 1))
    nct = jnp.transpose(new_xyz, (2, 0, 1))
    out = _pointhop_sc(gxt, nct)
    return out.T
